# Initial kernel scaffold; baseline (speedup 1.0000x reference)
import functools

import jax
import jax.numpy as jnp
import numpy as np
from jax import lax
from jax.experimental import pallas as pl
from jax.experimental.pallas import tpu as pltpu
from jax.experimental.pallas import tpu_sc as plsc

N = 10000
E = 320000
DIN = 128
DH = 128
DOUT = 40

NC = 2
NS = 16
NW = NC * NS
CH = 128
NCHUNK = 79
EPT = NCHUNK * CH
EPAD = NW * EPT
SROWS = 10112
RPT = SROWS // NS

BLK = 400
NBLK = N // BLK
BNC = float(1.0 / np.sqrt(1.0 + 1e-5))

_mesh = plsc.VectorSubcoreMesh(core_axis_name="c", subcore_axis_name="s")


def _zero_shared(shared, zbuf, s, width):
    z = jnp.zeros((16,), jnp.float32)

    def zrow(i, carry):
        for j in range(width // 16):
            zbuf[i, pl.ds(j * 16, 16)] = z
        return carry

    lax.fori_loop(0, CH, zrow, 0)
    a = s * RPT
    for off in (0, 128, 256, 384, 512):
        L = min(CH, RPT - off)
        pltpu.sync_copy(zbuf.at[pl.ds(0, L)], shared.at[pl.ds(a + off, L)])


def _make_agg(width, use_w, use_gather):
    scratch = [
        pltpu.VMEM_SHARED((SROWS, width), jnp.float32),
        pltpu.VMEM((CH,), jnp.int32),
        pltpu.VMEM((CH, width), jnp.float32),
        pltpu.SemaphoreType.DMA,
    ]
    if use_gather:
        scratch.append(pltpu.VMEM((CH,), jnp.int32))
    if use_w:
        scratch.append(pltpu.VMEM((CH,), jnp.float32))

    @functools.partial(
        pl.kernel,
        out_type=jax.ShapeDtypeStruct((NC, SROWS, width), jnp.float32),
        mesh=_mesh,
        scratch_types=scratch,
    )
    def body(*refs):
        if use_gather:
            t_hbm, row_hbm, col_hbm = refs[:3]
            rest = refs[3:]
            if use_w:
                w_hbm, out_hbm = rest[0], rest[1]
                rest = rest[2:]
            else:
                w_hbm = None
                out_hbm = rest[0]
                rest = rest[1:]
        else:
            vals_hbm, col_hbm, out_hbm = refs[0], refs[1], refs[2]
            rest = refs[3:]
        shared, cidx, rows, sem = rest[:4]
        ridx = rest[4] if use_gather else None
        wv = rest[5] if use_w else None

        c = lax.axis_index("c")
        s = lax.axis_index("s")
        wid = s * NC + c

        _zero_shared(shared, rows, s, width)
        plsc.subcore_barrier()

        def chunk(k, carry):
            base = wid * EPT + k * CH
            pltpu.sync_copy(col_hbm.at[pl.ds(base, CH)], cidx)
            if use_gather:
                pltpu.sync_copy(row_hbm.at[pl.ds(base, CH)], ridx)
                if use_w:
                    pltpu.sync_copy(w_hbm.at[pl.ds(base, CH)], wv)
                pltpu.async_copy(t_hbm.at[ridx], rows, sem).wait()
                if use_w:
                    def medge(e, cy):
                        eb = jnp.zeros((16,), jnp.int32) + e
                        wb = plsc.load_gather(wv, [eb])
                        for j in range(width // 16):
                            sl = pl.ds(j * 16, 16)
                            rows[e, sl] = rows[e, sl] * wb
                        return cy

                    lax.fori_loop(0, CH, medge, 0)
            else:
                pltpu.sync_copy(vals_hbm.at[pl.ds(base, CH)], rows)
            pltpu.sync_copy(rows, shared.at[cidx], add=True)
            return carry

        lax.fori_loop(0, NCHUNK, chunk, 0)
        plsc.subcore_barrier()
        a = s * RPT
        pltpu.sync_copy(shared.at[pl.ds(a, RPT)], out_hbm.at[c, pl.ds(a, RPT)])

    return body


_sc_deg = _make_agg(16, use_w=False, use_gather=False)
_sc_agg_w = _make_agg(DH, use_w=True, use_gather=True)
_sc_agg_1 = _make_agg(DH, use_w=False, use_gather=True)


def _col_pick(arr, col):
    ci = lax.broadcasted_iota(jnp.int32, arr.shape, 1)
    return jnp.sum(jnp.where(ci == col, arr, 0.0), axis=1)


def _tc1_body(x_ref, w1_ref, degp_ref, t1_ref, dis_ref):
    d = degp_ref[...]
    dsum = d[0] + d[1]
    degw = _col_pick(dsum, 0)
    deg1 = _col_pick(dsum, 1)
    disw = jnp.where(degw > 0, lax.rsqrt(degw), 0.0)
    dis1 = jnp.where(deg1 > 0, lax.rsqrt(deg1), 0.0)
    c8 = lax.broadcasted_iota(jnp.int32, (BLK, 8), 1)
    dis_ref[...] = jnp.where(
        c8 == 0, disw[:, None], jnp.where(c8 == 1, dis1[:, None], 0.0))
    t1_ref[...] = jnp.dot(
        x_ref[...], w1_ref[...], preferred_element_type=jnp.float32
    ) * disw[:, None]


def _mid_body(matmul, pp_ref, dis_ref, b_ref, g_ref, be_ref, w_ref, t_ref):
    p2 = pp_ref[...]
    p = p2[0] + p2[1]
    disw = _col_pick(dis_ref[...], 0)
    h = disw[:, None] * p + b_ref[...]
    h = h * BNC * g_ref[...] + be_ref[...]
    h = jnp.maximum(h, 0.0)
    if matmul:
        t_ref[...] = jnp.dot(
            h, w_ref[...], preferred_element_type=jnp.float32) * disw[:, None]
    else:
        dis1 = _col_pick(dis_ref[...], 1)
        t_ref[...] = h * dis1[:, None]


def _out_body(pp_ref, dis_ref, w3_ref, b3_ref, o_ref):
    p2 = pp_ref[...]
    p = p2[0] + p2[1]
    dis1 = _col_pick(dis_ref[...], 1)
    o_ref[...] = jnp.dot(
        dis1[:, None] * p, w3_ref[...], preferred_element_type=jnp.float32
    ) + b3_ref[...]


def _full_spec(shape):
    nd = len(shape)
    return pl.BlockSpec(shape, lambda i: (0,) * nd)


def kernel(x, edge_index, weight, W1, b1, gamma1, beta1, W2, b2, gamma2,
           beta2, W3, b3):
    f32 = jnp.float32
    row = edge_index[0]
    col = edge_index[1]
    pad = EPAD - E
    rowp = jnp.concatenate([row, jnp.zeros((pad,), jnp.int32)])
    colp = jnp.concatenate([col, jnp.full((pad,), N, jnp.int32)])
    wp = jnp.concatenate([weight, jnp.zeros((pad,), f32)])
    vals = jnp.zeros((EPAD, 16), f32).at[:, 0].set(wp).at[:E, 1].set(1.0)

    degp = _sc_deg(vals, colp)

    t1, dis = pl.pallas_call(
        _tc1_body,
        grid=(NBLK,),
        in_specs=[
            pl.BlockSpec((BLK, DIN), lambda i: (i, 0)),
            _full_spec((DIN, DH)),
            pl.BlockSpec((2, BLK, 16), lambda i: (0, i, 0)),
        ],
        out_specs=[
            pl.BlockSpec((BLK, DH), lambda i: (i, 0)),
            pl.BlockSpec((BLK, 8), lambda i: (i, 0)),
        ],
        out_shape=[
            jax.ShapeDtypeStruct((N, DH), f32),
            jax.ShapeDtypeStruct((N, 8), f32),
        ],
    )(x, W1, degp)

    p1 = _sc_agg_w(t1, rowp, colp, wp)

    def mid_call(body, has_w, out_d):
        return pl.pallas_call(
            body,
            grid=(NBLK,),
            in_specs=[
                pl.BlockSpec((2, BLK, DH), lambda i: (0, i, 0)),
                pl.BlockSpec((BLK, 8), lambda i: (i, 0)),
                _full_spec((1, DH)),
                _full_spec((1, DH)),
                _full_spec((1, DH)),
            ] + ([_full_spec((DH, DH))] if has_w else []),
            out_specs=pl.BlockSpec((BLK, out_d), lambda i: (i, 0)),
            out_shape=jax.ShapeDtypeStruct((N, out_d), f32),
        )

    t2 = mid_call(functools.partial(_mid_body, True), True, DH)(
        p1, dis, b1.reshape(1, DH), gamma1.reshape(1, DH),
        beta1.reshape(1, DH), W2)

    p2 = _sc_agg_w(t2, rowp, colp, wp)

    t3 = mid_call(
        lambda pp, d, b, g, be, t: _mid_body(False, pp, d, b, g, be, None, t),
        False, DH,
    )(p2, dis, b2.reshape(1, DH), gamma2.reshape(1, DH), beta2.reshape(1, DH))

    p3 = _sc_agg_1(t3, rowp, colp)

    out = pl.pallas_call(
        _out_body,
        grid=(NBLK,),
        in_specs=[
            pl.BlockSpec((2, BLK, DH), lambda i: (0, i, 0)),
            pl.BlockSpec((BLK, 8), lambda i: (i, 0)),
            _full_spec((DH, DOUT)),
            _full_spec((1, DOUT)),
        ],
        out_specs=pl.BlockSpec((BLK, DOUT), lambda i: (i, 0)),
        out_shape=jax.ShapeDtypeStruct((N, DOUT), f32),
    )(p3, dis, W3, b3.reshape(1, DOUT))

    return out


# speedup vs baseline: 9.4083x; 9.4083x over previous
import functools

import jax
import jax.numpy as jnp
import numpy as np
from jax import lax
from jax.experimental import pallas as pl
from jax.experimental.pallas import tpu as pltpu
from jax.experimental.pallas import tpu_sc as plsc

N = 10000
E = 320000
DIN = 128
DH = 128
DOUT = 40

NC = 2
NS = 16
NW = NC * NS
CH = 128
NCHUNK = 79
EPT = NCHUNK * CH
EPAD = NW * EPT
SROWS = 10112
RPT = SROWS // NS
NPAD = 10240

BLK = 400
NBLK = N // BLK
DBLK = 1280
BNC = float(1.0 / np.sqrt(1.0 + 1e-5))

_mesh = plsc.VectorSubcoreMesh(core_axis_name="c", subcore_axis_name="s")


@functools.partial(
    pl.kernel,
    out_type=jax.ShapeDtypeStruct((NW, 2, NPAD), jnp.float32),
    mesh=_mesh,
    compiler_params=pltpu.CompilerParams(needs_layout_passes=False),
    scratch_types=[
        pltpu.VMEM((NPAD,), jnp.float32),
        pltpu.VMEM((NPAD,), jnp.float32),
        pltpu.VMEM((1, CH), jnp.int32),
        pltpu.VMEM((CH,), jnp.float32),
    ],
)
def _sc_deg(col_hbm, w_hbm, out_hbm, hw, h1, cidx, wv):
    c = lax.axis_index("c")
    s = lax.axis_index("s")
    wid = s * NC + c

    z = jnp.zeros((16,), jnp.float32)
    ones = jnp.ones((16,), jnp.float32)

    def zrow(i, cy):
        hw[pl.ds(i * 16, 16)] = z
        h1[pl.ds(i * 16, 16)] = z
        return cy

    lax.fori_loop(0, NPAD // 16, zrow, 0)

    def chunk(k, cy):
        base = wid * EPT + k * CH
        pltpu.sync_copy(col_hbm.at[pl.ds(base, CH)], cidx.at[0])
        pltpu.sync_copy(w_hbm.at[pl.ds(base, CH)], wv)
        for g in range(CH // 16):
            cvec = cidx[0, pl.ds(g * 16, 16)]
            w16 = wv[pl.ds(g * 16, 16)]
            plsc.addupdate_scatter(hw, [cvec], w16)
            plsc.addupdate_scatter(h1, [cvec], ones)
        return cy

    lax.fori_loop(0, NCHUNK, chunk, 0)
    pltpu.sync_copy(hw, out_hbm.at[wid, 0])
    pltpu.sync_copy(h1, out_hbm.at[wid, 1])


def _make_agg(use_w):
    scratch = [
        pltpu.VMEM_SHARED((SROWS, DH), jnp.float32),
        pltpu.VMEM((1, CH), jnp.int32),
        pltpu.VMEM((CH,), jnp.int32),
        pltpu.VMEM((CH, DH), jnp.float32),
        pltpu.SemaphoreType.DMA,
    ]
    if use_w:
        scratch.append(pltpu.VMEM((CH,), jnp.float32))

    @functools.partial(
        pl.kernel,
        out_type=jax.ShapeDtypeStruct((NC, SROWS, DH), jnp.float32),
        mesh=_mesh,
        scratch_types=scratch,
    )
    def body(*refs):
        if use_w:
            (t_hbm, row_hbm, col_hbm, w_hbm, out_hbm,
             shared, cidx, ridx, rows, sem, wv) = refs
        else:
            (t_hbm, row_hbm, col_hbm, out_hbm,
             shared, cidx, ridx, rows, sem) = refs
            wv = None

        c = lax.axis_index("c")
        s = lax.axis_index("s")
        wid = s * NC + c

        z = jnp.zeros((16,), jnp.float32)

        def zrow(i, cy):
            for j in range(DH // 16):
                rows[i, pl.ds(j * 16, 16)] = z
            return cy

        lax.fori_loop(0, CH, zrow, 0)
        a = s * RPT
        for off in (0, 128, 256, 384, 512):
            L = min(CH, RPT - off)
            pltpu.sync_copy(rows.at[pl.ds(0, L)], shared.at[pl.ds(a + off, L)])
        plsc.subcore_barrier()

        def chunk(k, cy):
            base = wid * EPT + k * CH
            pltpu.sync_copy(col_hbm.at[pl.ds(base, CH)], cidx.at[0])
            pltpu.sync_copy(row_hbm.at[pl.ds(base, CH)], ridx)
            if use_w:
                pltpu.sync_copy(w_hbm.at[pl.ds(base, CH)], wv)
            pltpu.async_copy(t_hbm.at[ridx], rows, sem).wait()
            if use_w:
                def mgrp(g, cy2):
                    wb = wv[pl.ds(g * 16, 16)]
                    for l in range(16):
                        wsc = wb[l]
                        e = g * 16 + l
                        for j in range(DH // 16):
                            sl = pl.ds(j * 16, 16)
                            rows[e, sl] = rows[e, sl] * wsc
                    return cy2

                lax.fori_loop(0, CH // 16, mgrp, 0)
            pltpu.sync_copy(rows, shared.at[cidx.at[0]], add=True)
            return cy

        lax.fori_loop(0, NCHUNK, chunk, 0)
        plsc.subcore_barrier()
        pltpu.sync_copy(shared.at[pl.ds(a, RPT)], out_hbm.at[c, pl.ds(a, RPT)])

    return body


_sc_agg_w = _make_agg(True)
_sc_agg_1 = _make_agg(False)


def _deg_body(degp_ref, dis_ref):
    d = degp_ref[...]
    degw = jnp.sum(d[:, 0, :], axis=0)
    deg1 = jnp.sum(d[:, 1, :], axis=0)
    disw = jnp.where(degw > 0, lax.rsqrt(degw), 0.0)
    dis1 = jnp.where(deg1 > 0, lax.rsqrt(deg1), 0.0)
    r2 = lax.broadcasted_iota(jnp.int32, (2, DBLK), 0)
    dis_ref[...] = jnp.where(r2 == 0, disw[None, :], dis1[None, :])


def _col_pick(arr, col):
    ci = lax.broadcasted_iota(jnp.int32, arr.shape, 1)
    return jnp.sum(jnp.where(ci == col, arr, 0.0), axis=1)


def _tc1_body(x_ref, w1_ref, dis_ref, t1_ref):
    disw = _col_pick(dis_ref[...], 0)
    t1_ref[...] = jnp.dot(
        x_ref[...], w1_ref[...], preferred_element_type=jnp.float32
    ) * disw[:, None]


def _mid_body(matmul, pp_ref, dis_ref, b_ref, g_ref, be_ref, w_ref, t_ref):
    p2 = pp_ref[...]
    p = p2[0] + p2[1]
    disw = _col_pick(dis_ref[...], 0)
    h = disw[:, None] * p + b_ref[...]
    h = h * BNC * g_ref[...] + be_ref[...]
    h = jnp.maximum(h, 0.0)
    if matmul:
        t_ref[...] = jnp.dot(
            h, w_ref[...], preferred_element_type=jnp.float32) * disw[:, None]
    else:
        dis1 = _col_pick(dis_ref[...], 1)
        t_ref[...] = h * dis1[:, None]


def _out_body(pp_ref, dis_ref, w3_ref, b3_ref, o_ref):
    p2 = pp_ref[...]
    p = p2[0] + p2[1]
    dis1 = _col_pick(dis_ref[...], 1)
    o_ref[...] = jnp.dot(
        dis1[:, None] * p, w3_ref[...], preferred_element_type=jnp.float32
    ) + b3_ref[...]


def _full_spec(shape):
    nd = len(shape)
    return pl.BlockSpec(shape, lambda i: (0,) * nd)


def kernel(x, edge_index, weight, W1, b1, gamma1, beta1, W2, b2, gamma2,
           beta2, W3, b3):
    f32 = jnp.float32
    row = edge_index[0]
    col = edge_index[1]
    pad = EPAD - E
    rowp = jnp.concatenate([row, jnp.zeros((pad,), jnp.int32)])
    colp = jnp.concatenate([col, jnp.full((pad,), N, jnp.int32)])
    wp = jnp.concatenate([weight, jnp.zeros((pad,), f32)])

    degp = _sc_deg(colp, wp)

    dis_flat = pl.pallas_call(
        _deg_body,
        grid=(NPAD // DBLK,),
        in_specs=[pl.BlockSpec((NW, 2, DBLK), lambda i: (0, 0, i))],
        out_specs=pl.BlockSpec((2, DBLK), lambda i: (0, i)),
        out_shape=jax.ShapeDtypeStruct((2, NPAD), f32),
    )(degp)
    dis = dis_flat[:, :N].T

    t1 = pl.pallas_call(
        _tc1_body,
        grid=(NBLK,),
        in_specs=[
            pl.BlockSpec((BLK, DIN), lambda i: (i, 0)),
            _full_spec((DIN, DH)),
            pl.BlockSpec((BLK, 2), lambda i: (i, 0)),
        ],
        out_specs=pl.BlockSpec((BLK, DH), lambda i: (i, 0)),
        out_shape=jax.ShapeDtypeStruct((N, DH), f32),
    )(x, W1, dis)

    p1 = _sc_agg_w(t1, rowp, colp, wp)

    def mid_call(body, has_w, out_d):
        return pl.pallas_call(
            body,
            grid=(NBLK,),
            in_specs=[
                pl.BlockSpec((2, BLK, DH), lambda i: (0, i, 0)),
                pl.BlockSpec((BLK, 2), lambda i: (i, 0)),
                _full_spec((1, DH)),
                _full_spec((1, DH)),
                _full_spec((1, DH)),
            ] + ([_full_spec((DH, DH))] if has_w else []),
            out_specs=pl.BlockSpec((BLK, out_d), lambda i: (i, 0)),
            out_shape=jax.ShapeDtypeStruct((N, out_d), f32),
        )

    t2 = mid_call(functools.partial(_mid_body, True), True, DH)(
        p1, dis, b1.reshape(1, DH), gamma1.reshape(1, DH),
        beta1.reshape(1, DH), W2)

    p2 = _sc_agg_w(t2, rowp, colp, wp)

    t3 = mid_call(
        lambda pp, d, b, g, be, t: _mid_body(False, pp, d, b, g, be, None, t),
        False, DH,
    )(p2, dis, b2.reshape(1, DH), gamma2.reshape(1, DH), beta2.reshape(1, DH))

    p3 = _sc_agg_1(t3, rowp, colp)

    out = pl.pallas_call(
        _out_body,
        grid=(NBLK,),
        in_specs=[
            pl.BlockSpec((2, BLK, DH), lambda i: (0, i, 0)),
            pl.BlockSpec((BLK, 2), lambda i: (i, 0)),
            _full_spec((DH, DOUT)),
            _full_spec((1, DOUT)),
        ],
        out_specs=pl.BlockSpec((BLK, DOUT), lambda i: (i, 0)),
        out_shape=jax.ShapeDtypeStruct((N, DOUT), f32),
    )(p3, dis, W3, b3.reshape(1, DOUT))

    return out
